# baseline (speedup 1.0000x reference)
import functools

import jax
import jax.numpy as jnp
from jax.experimental import pallas as pl
from jax.experimental.pallas import tpu as pltpu

_LANES = 128
_OUT_W = 8


def _ring_kernel(self_hbm, nb_hbm, inv_ref, w_ref, out_ref,
                 self_buf, nb_buf, sems, *, n_nb, hidden, cb, n_chunks, depth):
    def nb_copy(c, s):
        return pltpu.make_async_copy(
            nb_hbm.at[pl.ds(c * cb, cb)], nb_buf.at[s], sems.at[0, s])

    def self_copy(c, s):
        return pltpu.make_async_copy(
            self_hbm.at[pl.ds(c * cb, cb)], self_buf.at[s], sems.at[1, s])

    for d in range(depth):
        nb_copy(d, d).start()
        self_copy(d, d).start()

    ws = w_ref[0:hidden, :]
    wn = w_ref[hidden:2 * hidden, :]
    b_row = w_ref[2 * hidden:2 * hidden + 1, :]

    def body(c, carry):
        s = jax.lax.rem(c, depth)
        nb_copy(c, s).wait()
        self_copy(c, s).wait()

        nb2 = nb_buf[s].reshape(cb * n_nb, hidden)
        acc_nb = jnp.dot(nb2, wn, preferred_element_type=jnp.float32)
        acc_nb = jnp.sum(acc_nb.reshape(cb, n_nb, _LANES), axis=1)

        acc = jnp.dot(self_buf[s], ws, preferred_element_type=jnp.float32)
        acc += acc_nb * inv_ref[pl.ds(c * cb, cb), :]
        acc += b_row
        out_ref[pl.ds(c * cb, cb), :] = acc[:, :_OUT_W]

        @pl.when(c + depth < n_chunks)
        def _():
            nb_copy(c + depth, s).start()
            self_copy(c + depth, s).start()
        return carry

    jax.lax.fori_loop(0, n_chunks, body, 0)


def kernel(emb_self, emb_nb, weight, bias, nb_counts):
    B, H = emb_self.shape
    _, N, _ = emb_nb.shape

    wt = weight.astype(jnp.float32)
    w_pack = jnp.zeros((2 * H + 8, _LANES), jnp.float32)
    w_pack = w_pack.at[0:2 * H, :2].set(wt.T)
    w_pack = w_pack.at[2 * H, :2].set(bias.astype(jnp.float32))

    inv_cnt = (1.0 / jnp.maximum(nb_counts.astype(jnp.float32), 1.0)
               ).reshape(B, 1)

    cb = 256 if B % 256 == 0 else B
    n_chunks = B // cb
    depth = min(4, n_chunks)

    out = pl.pallas_call(
        functools.partial(_ring_kernel, n_nb=N, hidden=H, cb=cb,
                          n_chunks=n_chunks, depth=depth),
        out_shape=jax.ShapeDtypeStruct((B, _OUT_W), jnp.float32),
        in_specs=[
            pl.BlockSpec(memory_space=pl.ANY),
            pl.BlockSpec(memory_space=pl.ANY),
            pl.BlockSpec(memory_space=pltpu.MemorySpace.VMEM),
            pl.BlockSpec(memory_space=pltpu.MemorySpace.VMEM),
        ],
        out_specs=pl.BlockSpec(memory_space=pltpu.MemorySpace.VMEM),
        scratch_shapes=[
            pltpu.VMEM((depth, cb, H), jnp.float32),
            pltpu.VMEM((depth, cb, N, H), jnp.float32),
            pltpu.SemaphoreType.DMA((2, depth)),
        ],
        compiler_params=pltpu.CompilerParams(vmem_limit_bytes=64 << 20),
        cost_estimate=pl.CostEstimate(
            flops=2 * B * H * _LANES * (1 + N // 2),
            transcendentals=0,
            bytes_accessed=B * (N + 1) * H * 4 + B * 4 + B * _OUT_W * 4
                           + (2 * H + 8) * _LANES * 4),
    )(emb_self, emb_nb, inv_cnt, w_pack)

    return out[:, :2]
